# Initial kernel scaffold; baseline (speedup 1.0000x reference)
import jax
import jax.numpy as jnp
from jax.experimental import pallas as pl


def kernel(inputs, embed_tables, cross_w, cross_b, W1, b1, W2, b2, W3, b3, Wo, bo):
    raise NotImplementedError("write your pallas kernel here")


# speedup vs baseline: 11.5763x; 11.5763x over previous
import jax
import jax.numpy as jnp
from jax import lax
from jax.experimental import pallas as pl
from jax.experimental.pallas import tpu as pltpu
from jax.experimental.pallas import tpu_sc as plsc

B = 4096
ND = 13
NF = 26
VOCAB = 1000
EMB = 32
SLOTS = NF + 2
DP = SLOTS * EMB
D = ND + NF * EMB
H1 = 1024
H2 = 1024
OUT_DIM = 256
NCROSS = 4
BM = 512


def _gather_x(table_aug, idx3, nw, nc, nchunk):
    mesh = plsc.VectorSubcoreMesh(core_axis_name="c", subcore_axis_name="s")

    def body(table_hbm, idx_hbm, x_hbm, idx_v, rows_v, sem):
        w = lax.axis_index("s") * nc + lax.axis_index("c")
        pltpu.sync_copy(idx_hbm.at[w], idx_v)

        def step(i, carry):
            cps = [
                pltpu.async_copy(
                    table_hbm.at[idx_v.at[i * 4 + u]], rows_v.at[i * 4 + u], sem
                )
                for u in range(4)
            ]
            for cp in cps:
                cp.wait()
            return carry

        lax.fori_loop(0, nchunk // 4, step, 0)
        pltpu.sync_copy(rows_v, x_hbm.at[w])

    k = pl.kernel(
        body,
        out_type=jax.ShapeDtypeStruct((nw, nchunk, 128, EMB), jnp.float32),
        mesh=mesh,
        compiler_params=pltpu.CompilerParams(use_tc_tiling_on_sc=False),
        scratch_types=[
            pltpu.VMEM((nchunk, 128), jnp.int32),
            pltpu.VMEM((nchunk, 128, EMB), jnp.float32),
            pltpu.SemaphoreType.DMA,
        ],
    )
    return k(table_aug, idx3)


def _dcn_tc(x_pad, cw_all, cw_p, cb_p, wox_t, W1p, b1, W2, b2, W3, b3, woh, bo):
    def body(x_ref, cwall_ref, cw_ref, cb_ref, wox_ref, w1_ref, b1_ref,
             w2_ref, b2_ref, w3_ref, b3_ref, woh_ref, bo_ref, out_ref):
        x = x_ref[...]
        P = jnp.dot(x, cwall_ref[...], preferred_element_type=jnp.float32)
        c = jnp.ones((BM, 1), jnp.float32)
        Bl = jnp.zeros((1, DP), jnp.float32)
        for l in range(NCROSS):
            beta = jnp.sum(Bl * cw_ref[l:l + 1, :])
            c = c + c * P[:, l:l + 1] + beta
            Bl = Bl + cb_ref[l:l + 1, :]
        gamma = jnp.sum(Bl * wox_ref[...])
        h = jnp.maximum(
            jnp.dot(x, w1_ref[...], preferred_element_type=jnp.float32)
            + b1_ref[...], 0.0)
        h = jnp.maximum(
            jnp.dot(h, w2_ref[...], preferred_element_type=jnp.float32)
            + b2_ref[...], 0.0)
        h = jnp.maximum(
            jnp.dot(h, w3_ref[...], preferred_element_type=jnp.float32)
            + b3_ref[...], 0.0)
        logit = (c * P[:, 4:5] + gamma
                 + jnp.dot(h, woh_ref[...], preferred_element_type=jnp.float32)
                 + bo_ref[0, 0])
        out_ref[...] = 1.0 / (1.0 + jnp.exp(-logit))

    def wspec(shape):
        return pl.BlockSpec(shape, lambda i: (0, 0))

    return pl.pallas_call(
        body,
        grid=(B // BM,),
        in_specs=[
            pl.BlockSpec((BM, DP), lambda i: (i, 0)),
            wspec((DP, NCROSS + 1)),
            wspec((NCROSS, DP)),
            wspec((NCROSS, DP)),
            wspec((1, DP)),
            wspec((DP, H1)),
            wspec((1, H1)),
            wspec((H1, H2)),
            wspec((1, H2)),
            wspec((H2, OUT_DIM)),
            wspec((1, OUT_DIM)),
            wspec((OUT_DIM, 1)),
            wspec((1, 1)),
        ],
        out_specs=pl.BlockSpec((BM, 1), lambda i: (i, 0)),
        out_shape=jax.ShapeDtypeStruct((B, 1), jnp.float32),
        compiler_params=pltpu.CompilerParams(
            dimension_semantics=("arbitrary",)),
    )(x_pad, cw_all, cw_p, cb_p, wox_t, W1p, b1, W2, b2, W3, b3, woh, bo)


def _pad_cols(m):
    k = m.shape[0]
    return jnp.concatenate([
        m[:, :ND], jnp.zeros((k, EMB - ND), jnp.float32),
        m[:, ND:], jnp.zeros((k, EMB), jnp.float32),
    ], axis=1)


def kernel(inputs, embed_tables, cross_w, cross_b, W1, b1, W2, b2, W3, b3, Wo, bo):
    info = plsc.get_sparse_core_info()
    nc, ns = info.num_cores, info.num_subcores
    nw = nc * ns
    nchunk = (B * SLOTS) // (nw * 128)

    dense = inputs[:, :ND]
    sp_idx = inputs[:, ND:].astype(jnp.int32)

    tables_flat = embed_tables.reshape(NF * VOCAB, EMB)
    dense_rows = jnp.pad(dense, ((0, 0), (0, EMB - ND)))
    zero_row = jnp.zeros((1, EMB), jnp.float32)
    table_aug = jnp.concatenate([tables_flat, dense_rows, zero_row], axis=0)

    base = NF * VOCAB
    emb_idx = sp_idx + (jnp.arange(NF, dtype=jnp.int32) * VOCAB)[None, :]
    dense_idx = base + jnp.arange(B, dtype=jnp.int32)[:, None]
    zero_idx = jnp.full((B, 1), base + B, jnp.int32)
    idx = jnp.concatenate([dense_idx, emb_idx, zero_idx], axis=1)
    idx3 = idx.reshape(nw, nchunk, 128)

    x4 = _gather_x(table_aug, idx3, nw, nc, nchunk)
    x_pad = x4.reshape(B, DP)

    W1p = jnp.concatenate([
        W1[:ND], jnp.zeros((EMB - ND, H1), jnp.float32),
        W1[ND:], jnp.zeros((EMB, H1), jnp.float32),
    ], axis=0)
    cw_p = _pad_cols(cross_w)
    cb_p = _pad_cols(cross_b)
    wox_t = _pad_cols(Wo[:D].reshape(1, D))
    cw_all = jnp.concatenate([cw_p, wox_t], axis=0).T
    woh = Wo[D:]

    return _dcn_tc(
        x_pad, cw_all, cw_p, cb_p, wox_t, W1p,
        b1.reshape(1, H1), W2, b2.reshape(1, H2), W3, b3.reshape(1, OUT_DIM),
        woh, bo.reshape(1, 1))


# speedup vs baseline: 13.1693x; 1.1376x over previous
import jax
import jax.numpy as jnp
from jax import lax
from jax.experimental import pallas as pl
from jax.experimental.pallas import tpu as pltpu
from jax.experimental.pallas import tpu_sc as plsc

B = 4096
ND = 13
NF = 26
VOCAB = 1000
EMB = 32
SLOTS = NF + 2
DP = SLOTS * EMB
D = ND + NF * EMB
H1 = 1024
H2 = 1024
OUT_DIM = 256
NCROSS = 4
BM = 512


def _gather_x(table_aug, idx3, nw, nc, nchunk):
    mesh = plsc.VectorSubcoreMesh(core_axis_name="c", subcore_axis_name="s")

    def body(table_hbm, idx_hbm, x_hbm, idx_v, rows_v, sem):
        w = lax.axis_index("s") * nc + lax.axis_index("c")
        pltpu.sync_copy(idx_hbm.at[w], idx_v)

        def step(i, carry):
            cps = [
                pltpu.async_copy(
                    table_hbm.at[idx_v.at[i * 4 + u]], rows_v.at[i * 4 + u], sem
                )
                for u in range(4)
            ]
            for cp in cps:
                cp.wait()
            return carry

        lax.fori_loop(0, nchunk // 4, step, 0)
        pltpu.sync_copy(rows_v, x_hbm.at[w])

    k = pl.kernel(
        body,
        out_type=jax.ShapeDtypeStruct((nw, nchunk, 128, EMB), jnp.bfloat16),
        mesh=mesh,
        compiler_params=pltpu.CompilerParams(use_tc_tiling_on_sc=False),
        scratch_types=[
            pltpu.VMEM((nchunk, 128), jnp.int32),
            pltpu.VMEM((nchunk, 128, EMB), jnp.bfloat16),
            pltpu.SemaphoreType.DMA,
        ],
    )
    return k(table_aug, idx3)


def _dcn_tc(x_pad, cw_all, cw_p, cb_p, wox_t, W1p, b1, W2, b2, W3, b3, woh, bo):
    def body(x_ref, cwall_ref, cw_ref, cb_ref, wox_ref, w1_ref, b1_ref,
             w2_ref, b2_ref, w3_ref, b3_ref, woh_ref, bo_ref, out_ref):
        x = x_ref[...]
        P = jnp.dot(x, cwall_ref[...], preferred_element_type=jnp.float32)
        c = jnp.ones((BM, 1), jnp.float32)
        Bl = jnp.zeros((1, DP), jnp.float32)
        for l in range(NCROSS):
            beta = jnp.sum(Bl * cw_ref[l:l + 1, :])
            c = c + c * P[:, l:l + 1] + beta
            Bl = Bl + cb_ref[l:l + 1, :]
        gamma = jnp.sum(Bl * wox_ref[...])
        h = jnp.maximum(
            jnp.dot(x, w1_ref[...], preferred_element_type=jnp.float32)
            + b1_ref[...], 0.0).astype(jnp.bfloat16)
        h = jnp.maximum(
            jnp.dot(h, w2_ref[...], preferred_element_type=jnp.float32)
            + b2_ref[...], 0.0).astype(jnp.bfloat16)
        h = jnp.maximum(
            jnp.dot(h, w3_ref[...], preferred_element_type=jnp.float32)
            + b3_ref[...], 0.0).astype(jnp.bfloat16)
        logit = (c * P[:, 4:5] + gamma
                 + jnp.dot(h, woh_ref[...], preferred_element_type=jnp.float32)
                 + bo_ref[0, 0])
        out_ref[...] = 1.0 / (1.0 + jnp.exp(-logit))

    def wspec(shape):
        return pl.BlockSpec(shape, lambda i: (0, 0))

    return pl.pallas_call(
        body,
        grid=(B // BM,),
        in_specs=[
            pl.BlockSpec((BM, DP), lambda i: (i, 0)),
            wspec((DP, NCROSS + 1)),
            wspec((NCROSS, DP)),
            wspec((NCROSS, DP)),
            wspec((1, DP)),
            wspec((DP, H1)),
            wspec((1, H1)),
            wspec((H1, H2)),
            wspec((1, H2)),
            wspec((H2, OUT_DIM)),
            wspec((1, OUT_DIM)),
            wspec((OUT_DIM, 1)),
            wspec((1, 1)),
        ],
        out_specs=pl.BlockSpec((BM, 1), lambda i: (i, 0)),
        out_shape=jax.ShapeDtypeStruct((B, 1), jnp.float32),
        compiler_params=pltpu.CompilerParams(
            dimension_semantics=("arbitrary",)),
    )(x_pad, cw_all, cw_p, cb_p, wox_t, W1p, b1, W2, b2, W3, b3, woh, bo)


def _pad_cols(m):
    k = m.shape[0]
    return jnp.concatenate([
        m[:, :ND], jnp.zeros((k, EMB - ND), jnp.float32),
        m[:, ND:], jnp.zeros((k, EMB), jnp.float32),
    ], axis=1)


def kernel(inputs, embed_tables, cross_w, cross_b, W1, b1, W2, b2, W3, b3, Wo, bo):
    info = plsc.get_sparse_core_info()
    nc, ns = info.num_cores, info.num_subcores
    nw = nc * ns
    nchunk = (B * SLOTS) // (nw * 128)

    dense = inputs[:, :ND]
    sp_idx = inputs[:, ND:].astype(jnp.int32)

    bf16 = jnp.bfloat16
    tables_flat = embed_tables.astype(bf16).reshape(NF * VOCAB, EMB)
    dense_rows = jnp.pad(dense.astype(bf16), ((0, 0), (0, EMB - ND)))
    zero_row = jnp.zeros((1, EMB), bf16)
    table_aug = jnp.concatenate([tables_flat, dense_rows, zero_row], axis=0)

    base = NF * VOCAB
    emb_idx = sp_idx + (jnp.arange(NF, dtype=jnp.int32) * VOCAB)[None, :]
    dense_idx = base + jnp.arange(B, dtype=jnp.int32)[:, None]
    zero_idx = jnp.full((B, 1), base + B, jnp.int32)
    idx = jnp.concatenate([dense_idx, emb_idx, zero_idx], axis=1)
    idx3 = idx.reshape(nw, nchunk, 128)

    x4 = _gather_x(table_aug, idx3, nw, nc, nchunk)
    x_pad = x4.reshape(B, DP)

    W1p = jnp.concatenate([
        W1.astype(bf16)[:ND], jnp.zeros((EMB - ND, H1), bf16),
        W1.astype(bf16)[ND:], jnp.zeros((EMB, H1), bf16),
    ], axis=0)
    cw_p = _pad_cols(cross_w)
    cb_p = _pad_cols(cross_b)
    wox_t = _pad_cols(Wo[:D].reshape(1, D))
    cw_all = jnp.concatenate([cw_p, wox_t], axis=0).T.astype(bf16)
    woh = Wo[D:].astype(bf16)

    return _dcn_tc(
        x_pad, cw_all, cw_p, cb_p, wox_t, W1p,
        b1.reshape(1, H1), W2.astype(bf16), b2.reshape(1, H2),
        W3.astype(bf16), b3.reshape(1, OUT_DIM),
        woh, bo.reshape(1, 1))
